# baseline (speedup 1.0000x reference)
import functools

import jax
import jax.numpy as jnp
from jax import lax
from jax.experimental import pallas as pl
from jax.experimental.pallas import tpu as pltpu
from jax.experimental.pallas import tpu_sc as plsc

NC = 2
NS = 16
NW = NC * NS
L = 16


def _body(T, N, D, rows_per_tile, n_chunks, hid_hbm, cu_hbm, out_hbm,
          cu_v, buf0, buf1, buf2, gsem0, gsem1, gsem2,
          ssem0, ssem1, ssem2):
    wid = lax.axis_index("s") * NC + lax.axis_index("c")
    base = wid * rows_per_tile
    bufs = (buf0, buf1, buf2)
    gsems = (gsem0, gsem1, gsem2)
    ssems = (ssem0, ssem1, ssem2)

    def dst_of(g):
        q = base + jnp.int32(g * L) + lax.iota(jnp.int32, L)
        s = jnp.zeros((L,), jnp.int32)
        steps = []
        st = 1
        while st < N:
            steps.append(st)
            st *= 2
        for step in reversed(steps):
            cand = jnp.minimum(s + jnp.int32(step), jnp.int32(N - 1))
            v = plsc.load_gather(cu_v, [cand])
            s = jnp.where(v <= q, cand, s)
        return (jnp.int32(T) - plsc.load_gather(cu_v, [s + 1])
                + q - plsc.load_gather(cu_v, [s]))

    def read_chunk(g, b):
        pltpu.async_copy(hid_hbm.at[pl.ds(base + g * L, L)], bufs[b],
                         gsems[b])

    def wait_read(b):
        pltpu.make_async_copy(hid_hbm.at[pl.ds(0, L)], bufs[b],
                              gsems[b]).wait()

    sd = [None] * n_chunks

    def scatter(g):
        b = g % 3
        d = dst_of(g)
        wait_read(b)
        sd[g] = pltpu.async_copy(bufs[b], out_hbm.at[d], ssems[b])

    for g in range(3):
        read_chunk(g, g)
    pltpu.sync_copy(cu_hbm, cu_v)

    for g in range(n_chunks):
        b = g % 3
        if g >= 3:
            sd[g - 3].wait()
            read_chunk(g, b)
        if g >= 1:
            scatter(g - 1)
    scatter(n_chunks - 1)
    for g in range(n_chunks - 2, n_chunks):
        sd[g].wait()


def kernel(hidden_states, cu_seqlens):
    T, D = hidden_states.shape
    N = cu_seqlens.shape[0] - 1
    rows_per_tile = T // NW
    n_chunks = rows_per_tile // L

    mesh = plsc.VectorSubcoreMesh(core_axis_name="c", subcore_axis_name="s")
    body = functools.partial(_body, T, N, D, rows_per_tile, n_chunks)
    f = pl.kernel(
        body,
        out_type=jax.ShapeDtypeStruct((T, D), jnp.float32),
        mesh=mesh,
        compiler_params=pltpu.CompilerParams(needs_layout_passes=False),
        scratch_types=[
            pltpu.VMEM((N + 1,), jnp.int32),
            pltpu.VMEM((L, D), jnp.float32),
            pltpu.VMEM((L, D), jnp.float32),
            pltpu.VMEM((L, D), jnp.float32),
            pltpu.SemaphoreType.DMA,
            pltpu.SemaphoreType.DMA,
            pltpu.SemaphoreType.DMA,
            pltpu.SemaphoreType.DMA,
            pltpu.SemaphoreType.DMA,
            pltpu.SemaphoreType.DMA,
        ],
    )
    return f(hidden_states, cu_seqlens.astype(jnp.int32))


# speedup vs baseline: 1.0047x; 1.0047x over previous
import functools

import jax
import jax.numpy as jnp
from jax import lax
from jax.experimental import pallas as pl
from jax.experimental.pallas import tpu as pltpu
from jax.experimental.pallas import tpu_sc as plsc

NC = 2
NS = 16
NW = NC * NS
L = 16


def _body(T, N, D, rows_per_tile, n_chunks, hid_hbm, cu_hbm, out_hbm,
          cu_v, buf0, buf1, buf2, gsem0, gsem1, gsem2,
          ssem0, ssem1, ssem2):
    wid = lax.axis_index("s") * NC + lax.axis_index("c")
    base = wid * rows_per_tile
    bufs = (buf0, buf1, buf2)
    gsems = (gsem0, gsem1, gsem2)
    ssems = (ssem0, ssem1, ssem2)

    def dst_of(g):
        q = base + jnp.int32(g * L) + lax.iota(jnp.int32, L)
        s = jnp.zeros((L,), jnp.int32)
        steps = []
        st = 1
        while st < N:
            steps.append(st)
            st *= 2
        for step in reversed(steps):
            cand = jnp.minimum(s + jnp.int32(step), jnp.int32(N - 1))
            v = plsc.load_gather(cu_v, [cand])
            s = jnp.where(v <= q, cand, s)
        return (jnp.int32(T) - plsc.load_gather(cu_v, [s + 1])
                + q - plsc.load_gather(cu_v, [s]))

    def read_chunk(g, b):
        pltpu.async_copy(hid_hbm.at[pl.ds(base + g * L, L)], bufs[b],
                         gsems[b])

    def wait_read(b):
        pltpu.make_async_copy(hid_hbm.at[pl.ds(0, L)], bufs[b],
                              gsems[b]).wait()

    sd = [None] * n_chunks

    def scatter(g):
        b = g % 3
        d = dst_of(g)
        wait_read(b)
        sd[g] = pltpu.async_copy(bufs[b], out_hbm.at[d], ssems[b])

    for g in range(3):
        read_chunk(g, g)
    pltpu.sync_copy(cu_hbm, cu_v)

    for g in range(n_chunks):
        b = g % 3
        if g >= 3:
            sd[g - 3].wait()
            read_chunk(g, b)
        if g >= 1:
            scatter(g - 1)
    scatter(n_chunks - 1)
    for g in range(n_chunks - 3, n_chunks):
        sd[g].wait()


def kernel(hidden_states, cu_seqlens):
    T, D = hidden_states.shape
    N = cu_seqlens.shape[0] - 1
    rows_per_tile = T // NW
    n_chunks = rows_per_tile // L

    mesh = plsc.VectorSubcoreMesh(core_axis_name="c", subcore_axis_name="s")
    body = functools.partial(_body, T, N, D, rows_per_tile, n_chunks)
    f = pl.kernel(
        body,
        out_type=jax.ShapeDtypeStruct((T, D), jnp.float32),
        mesh=mesh,
        compiler_params=pltpu.CompilerParams(needs_layout_passes=False),
        scratch_types=[
            pltpu.VMEM((N + 1,), jnp.int32),
            pltpu.VMEM((L, D), jnp.float32),
            pltpu.VMEM((L, D), jnp.float32),
            pltpu.VMEM((L, D), jnp.float32),
            pltpu.SemaphoreType.DMA,
            pltpu.SemaphoreType.DMA,
            pltpu.SemaphoreType.DMA,
            pltpu.SemaphoreType.DMA,
            pltpu.SemaphoreType.DMA,
            pltpu.SemaphoreType.DMA,
        ],
    )
    return f(hidden_states, cu_seqlens.astype(jnp.int32))
